# baseline (speedup 1.0000x reference)
import functools

import jax
import jax.numpy as jnp
from jax import lax
from jax.experimental import pallas as pl
from jax.experimental.pallas import tpu as pltpu
from jax.experimental.pallas import tpu_sc as plsc

N = 1048576
D = 32
OUT = 16
LANES = 16
NUM_CORES = 2
NUM_SUBCORES = 16
NW = NUM_CORES * NUM_SUBCORES

N_SC = 557056
N_TC = N - N_SC
ROWS_W = N_SC // NW
CW = 512
NCHUNK = ROWS_W // CW
GROUPS = CW // LANES

TC_BC = 16384
TC_OFF = N_SC // TC_BC

_mesh = plsc.VectorSubcoreMesh(core_axis_name="c", subcore_axis_name="s")


@functools.partial(
    pl.kernel,
    out_type=jax.ShapeDtypeStruct((N_SC,), jnp.float32),
    mesh=_mesh,
    scratch_types=[
        pltpu.VMEM((D, CW), jnp.float32),
        pltpu.VMEM((D, CW), jnp.float32),
        pltpu.VMEM((ROWS_W,), jnp.float32),
        pltpu.VMEM((OUT * D,), jnp.float32),
        pltpu.VMEM((OUT,), jnp.float32),
        pltpu.SemaphoreType.DMA,
        pltpu.SemaphoreType.DMA,
    ],
    compiler_params=pltpu.CompilerParams(
        needs_layout_passes=False, use_tc_tiling_on_sc=True),
)
def _sc_rowsum(xt_hbm, w_hbm, b_hbm, out_hbm,
               buf0, buf1, obuf, wvm, bvm, sem0, sem1):
    wid = lax.axis_index("s") * NUM_CORES + lax.axis_index("c")
    row0 = wid * ROWS_W

    def in_copy(c, buf, sem):
        return pltpu.make_async_copy(
            xt_hbm.at[:, pl.ds(row0 + c * CW, CW)], buf, sem)

    in_copy(0, buf0, sem0).start()
    in_copy(1, buf1, sem1).start()

    pltpu.sync_copy(w_hbm, wvm)
    pltpu.sync_copy(b_hbm, bvm)
    bvec = bvm[...]
    bsum = bvec[0]
    for o in range(1, OUT):
        bsum = bsum + bvec[o]
    wj = []
    for h in range(D // LANES):
        acc = wvm[pl.ds(h * LANES, LANES)]
        for o in range(1, OUT):
            acc = acc + wvm[pl.ds(o * D + h * LANES, LANES)]
        wj.extend(acc[j] for j in range(LANES))
    wjb = [jnp.broadcast_to(w, (LANES,)) for w in wj]
    bsum_vec = jnp.broadcast_to(bsum, (LANES,))

    def chunk_compute(c, buf):
        @plsc.parallel_loop(0, GROUPS, unroll=2)
        def gbody(g):
            r = g * LANES
            accs = [None] * 4
            for j in range(D):
                t = buf[j, pl.ds(r, LANES)] * wjb[j]
                k = j % 4
                accs[k] = t if accs[k] is None else accs[k] + t
            acc = (accs[0] + accs[1]) + (accs[2] + accs[3]) + bsum_vec
            obuf[pl.ds(c * CW + r, LANES)] = acc

    def pair_body(i, carry):
        for par, (buf, sem) in enumerate(((buf0, sem0), (buf1, sem1))):
            c = i * 2 + par
            in_copy(c, buf, sem).wait()
            chunk_compute(c, buf)

            @pl.when(i < NCHUNK // 2 - 1)
            def _():
                in_copy(c + 2, buf, sem).start()
        return carry

    lax.fori_loop(0, NCHUNK // 2, pair_body, 0)

    pltpu.sync_copy(obuf, out_hbm.at[pl.ds(row0, ROWS_W)])


def _tc_body(xt_ref, w_ref, b_ref, o_ref):
    wsum = jnp.sum(w_ref[...], axis=0)
    bsum = jnp.sum(b_ref[...])
    blk = xt_ref[...]
    o_ref[...] = jnp.sum(blk * wsum[:, None], axis=0) + bsum


_tc_rowsum = pl.pallas_call(
    _tc_body,
    grid=(N_TC // TC_BC,),
    in_specs=[
        pl.BlockSpec((D, TC_BC), lambda i: (0, TC_OFF + i)),
        pl.BlockSpec((OUT, D), lambda i: (0, 0)),
        pl.BlockSpec((OUT,), lambda i: (0,)),
    ],
    out_specs=pl.BlockSpec((TC_BC,), lambda i: (i,)),
    out_shape=jax.ShapeDtypeStruct((N_TC,), jnp.float32),
)


def kernel(x, W, b):
    xt = x.T
    s_sc = _sc_rowsum(xt, W.reshape(-1), b)
    s_tc = _tc_rowsum(xt, W, b)
    return jnp.concatenate([s_sc, s_tc])


# speedup vs baseline: 1.0258x; 1.0258x over previous
import functools

import jax
import jax.numpy as jnp
from jax import lax
from jax.experimental import pallas as pl
from jax.experimental.pallas import tpu as pltpu
from jax.experimental.pallas import tpu_sc as plsc

N = 1048576
D = 32
OUT = 16
LANES = 16
NUM_CORES = 2
NUM_SUBCORES = 16
NW = NUM_CORES * NUM_SUBCORES

N_SC = 589824
N_TC = N - N_SC
ROWS_W = N_SC // NW
CW = 1024
NCHUNK = ROWS_W // CW
GROUPS = CW // LANES

TC_BC = 16384
TC_OFF = N_SC // TC_BC

_mesh = plsc.VectorSubcoreMesh(core_axis_name="c", subcore_axis_name="s")


@functools.partial(
    pl.kernel,
    out_type=jax.ShapeDtypeStruct((N_SC,), jnp.float32),
    mesh=_mesh,
    scratch_types=[
        pltpu.VMEM((D, CW), jnp.float32),
        pltpu.VMEM((D, CW), jnp.float32),
        pltpu.VMEM((ROWS_W,), jnp.float32),
        pltpu.VMEM((OUT * D,), jnp.float32),
        pltpu.VMEM((OUT,), jnp.float32),
        pltpu.SemaphoreType.DMA,
        pltpu.SemaphoreType.DMA,
    ],
    compiler_params=pltpu.CompilerParams(
        needs_layout_passes=False, use_tc_tiling_on_sc=True),
)
def _sc_rowsum(xt_hbm, w_hbm, b_hbm, out_hbm,
               buf0, buf1, obuf, wvm, bvm, sem0, sem1):
    wid = lax.axis_index("s") * NUM_CORES + lax.axis_index("c")
    row0 = wid * ROWS_W

    def in_copy(c, buf, sem):
        return pltpu.make_async_copy(
            xt_hbm.at[:, pl.ds(row0 + c * CW, CW)], buf, sem)

    in_copy(0, buf0, sem0).start()
    in_copy(1, buf1, sem1).start()

    pltpu.sync_copy(w_hbm, wvm)
    pltpu.sync_copy(b_hbm, bvm)
    bvec = bvm[...]
    bsum = bvec[0]
    for o in range(1, OUT):
        bsum = bsum + bvec[o]
    wj = []
    for h in range(D // LANES):
        acc = wvm[pl.ds(h * LANES, LANES)]
        for o in range(1, OUT):
            acc = acc + wvm[pl.ds(o * D + h * LANES, LANES)]
        wj.extend(acc[j] for j in range(LANES))
    wjb = [jnp.broadcast_to(w, (LANES,)) for w in wj]
    bsum_vec = jnp.broadcast_to(bsum, (LANES,))

    def chunk_compute(c, buf):
        @plsc.parallel_loop(0, GROUPS, unroll=2)
        def gbody(g):
            r = g * LANES
            accs = [None] * 4
            for j in range(D):
                t = buf[j, pl.ds(r, LANES)] * wjb[j]
                k = j % 4
                accs[k] = t if accs[k] is None else accs[k] + t
            acc = (accs[0] + accs[1]) + (accs[2] + accs[3]) + bsum_vec
            obuf[pl.ds(c * CW + r, LANES)] = acc

    def pair_body(i, carry):
        for par, (buf, sem) in enumerate(((buf0, sem0), (buf1, sem1))):
            c = i * 2 + par
            in_copy(c, buf, sem).wait()
            chunk_compute(c, buf)

            @pl.when(i < NCHUNK // 2 - 1)
            def _():
                in_copy(c + 2, buf, sem).start()
        return carry

    lax.fori_loop(0, NCHUNK // 2, pair_body, 0)

    pltpu.sync_copy(obuf, out_hbm.at[pl.ds(row0, ROWS_W)])


def _tc_body(xt_ref, w_ref, b_ref, o_ref):
    wsum = jnp.sum(w_ref[...], axis=0)
    bsum = jnp.sum(b_ref[...])
    blk = xt_ref[...]
    o_ref[...] = jnp.sum(blk * wsum[:, None], axis=0) + bsum


_tc_rowsum = pl.pallas_call(
    _tc_body,
    grid=(N_TC // TC_BC,),
    in_specs=[
        pl.BlockSpec((D, TC_BC), lambda i: (0, TC_OFF + i)),
        pl.BlockSpec((OUT, D), lambda i: (0, 0)),
        pl.BlockSpec((OUT,), lambda i: (0,)),
    ],
    out_specs=pl.BlockSpec((TC_BC,), lambda i: (i,)),
    out_shape=jax.ShapeDtypeStruct((N_TC,), jnp.float32),
)


def kernel(x, W, b):
    xt = x.T
    s_sc = _sc_rowsum(xt, W.reshape(-1), b)
    s_tc = _tc_rowsum(xt, W, b)
    return jnp.concatenate([s_sc, s_tc])
